# baseline (speedup 1.0000x reference)
import functools

import jax
import jax.numpy as jnp
from jax import lax
from jax.experimental import pallas as pl
from jax.experimental.pallas import tpu as pltpu
from jax.experimental.pallas import tpu_sc as plsc


def _sc_gather_sum(inputs, emb_table):
    C, B = inputs.shape
    D = emb_table.shape[1]
    info = plsc.get_sparse_core_info()
    nw = info.num_cores * info.num_subcores
    b_per_w = B // nw
    mesh = plsc.VectorSubcoreMesh(core_axis_name="c", subcore_axis_name="s")

    @functools.partial(
        pl.kernel,
        mesh=mesh,
        compiler_params=pltpu.CompilerParams(use_tc_tiling_on_sc=False),
        out_type=jax.ShapeDtypeStruct((B, D), jnp.float32),
        scratch_types=[
            pltpu.VMEM((C * b_per_w,), jnp.int32),
            pltpu.VMEM((C * b_per_w, D), jnp.float32),
            pltpu.VMEM((b_per_w, D), jnp.float32),
            pltpu.SemaphoreType.DMA,
        ],
    )
    def k(idx_hbm, table_hbm, out_hbm, idx_v, rows_v, acc_v, sem):
        cid = lax.axis_index("c")
        sid = lax.axis_index("s")
        wid = sid * info.num_cores + cid
        base = wid * b_per_w
        for c in range(C):
            pltpu.sync_copy(
                idx_hbm.at[c, pl.ds(base, b_per_w)],
                idx_v.at[pl.ds(c * b_per_w, b_per_w)],
            )
        pltpu.async_copy(table_hbm.at[idx_v], rows_v, sem).wait()

        def body(i, carry):
            for j in range(D // 16):
                v = rows_v[i, pl.ds(j * 16, 16)]
                for c in range(1, C):
                    v = v + rows_v[i + c * b_per_w, pl.ds(j * 16, 16)]
                acc_v[i, pl.ds(j * 16, 16)] = v
            return carry

        lax.fori_loop(0, b_per_w, body, 0)
        pltpu.sync_copy(acc_v, out_hbm.at[pl.ds(base, b_per_w)])

    return k(inputs, emb_table)


def _tc_matmul_t(emb_aug, wt, b2):
    B, K1 = emb_aug.shape
    V = wt.shape[1]
    K = wt.shape[0]
    TV = 2048
    grid = (V + TV - 1) // TV

    def mm(w_ref, b_ref, emb_ref, out_ref):
        w65 = jnp.concatenate([w_ref[...], b_ref[...]], axis=0)
        out_ref[...] = lax.dot_general(
            w65,
            emb_ref[...],
            (((0,), (1,)), ((), ())),
            preferred_element_type=jnp.float32,
        )

    return pl.pallas_call(
        mm,
        grid=(grid,),
        in_specs=[
            pl.BlockSpec((K, TV), lambda i: (0, i)),
            pl.BlockSpec((1, TV), lambda i: (0, i)),
            pl.BlockSpec((B, K1), lambda i: (0, 0)),
        ],
        out_specs=pl.BlockSpec((TV, B), lambda i: (i, 0)),
        out_shape=jax.ShapeDtypeStruct((V, B), jnp.float32),
    )(wt, b2, emb_aug)


def kernel(inputs, emb_table, W, b):
    embeds = _sc_gather_sum(inputs.astype(jnp.int32), emb_table)
    emb_aug = jnp.concatenate(
        [embeds, jnp.ones((embeds.shape[0], 1), jnp.float32)], axis=1)
    out_t = _tc_matmul_t(emb_aug, W.T, b.reshape(1, -1))
    return out_t.T


# speedup vs baseline: 1.0609x; 1.0609x over previous
import functools

import jax
import jax.numpy as jnp
from jax import lax
from jax.experimental import pallas as pl
from jax.experimental.pallas import tpu as pltpu
from jax.experimental.pallas import tpu_sc as plsc


def _sc_gather_sum(inputs, table128, D):
    C, B = inputs.shape
    DP = table128.shape[1]
    info = plsc.get_sparse_core_info()
    nw = info.num_cores * info.num_subcores
    b_per_w = B // nw
    mesh = plsc.VectorSubcoreMesh(core_axis_name="c", subcore_axis_name="s")

    @functools.partial(
        pl.kernel,
        mesh=mesh,
        compiler_params=pltpu.CompilerParams(use_tc_tiling_on_sc=False),
        out_type=jax.ShapeDtypeStruct((B, D), jnp.float32),
        scratch_types=[
            pltpu.VMEM((C * b_per_w,), jnp.int32),
            pltpu.VMEM((C * b_per_w, DP), jnp.float32),
            pltpu.VMEM((b_per_w, D), jnp.float32),
            pltpu.SemaphoreType.DMA,
        ],
    )
    def k(idx_hbm, table_hbm, out_hbm, idx_v, rows_v, acc_v, sem):
        cid = lax.axis_index("c")
        sid = lax.axis_index("s")
        wid = sid * info.num_cores + cid
        base = wid * b_per_w
        for c in range(C):
            pltpu.sync_copy(
                idx_hbm.at[c, pl.ds(base, b_per_w)],
                idx_v.at[pl.ds(c * b_per_w, b_per_w)],
            )
        pltpu.async_copy(table_hbm.at[idx_v], rows_v, sem).wait()

        def body(i, carry):
            for j in range(D // 16):
                v = rows_v[i, pl.ds(j * 16, 16)]
                for c in range(1, C):
                    v = v + rows_v[i + c * b_per_w, pl.ds(j * 16, 16)]
                acc_v[i, pl.ds(j * 16, 16)] = v
            return carry

        lax.fori_loop(0, b_per_w, body, 0)
        pltpu.sync_copy(acc_v, out_hbm.at[pl.ds(base, b_per_w)])

    return k(inputs, table128)


def _tc_transpose_pad(emb_t, DP):
    K, V = emb_t.shape
    TV = 2048
    grid = (V + TV - 1) // TV
    eye = jnp.eye(K, dtype=jnp.float32)

    def tp(w_ref, eye_ref, out_ref):
        t = lax.dot_general(
            w_ref[...],
            eye_ref[...],
            (((0,), (0,)), ((), ())),
            preferred_element_type=jnp.float32,
        )
        out_ref[...] = jnp.concatenate(
            [t, jnp.zeros((t.shape[0], DP - K), jnp.float32)], axis=1)

    return pl.pallas_call(
        tp,
        grid=(grid,),
        in_specs=[
            pl.BlockSpec((K, TV), lambda i: (0, i)),
            pl.BlockSpec((K, K), lambda i: (0, 0)),
        ],
        out_specs=pl.BlockSpec((TV, DP), lambda i: (i, 0)),
        out_shape=jax.ShapeDtypeStruct((V, DP), jnp.float32),
    )(emb_t, eye)


def _tc_matmul_t(emb_aug, wt, b2):
    B, K1 = emb_aug.shape
    V = wt.shape[1]
    K = wt.shape[0]
    TV = 2048
    grid = (V + TV - 1) // TV

    def mm(w_ref, b_ref, emb_ref, out_ref):
        w65 = jnp.concatenate([w_ref[...], b_ref[...]], axis=0)
        out_ref[...] = lax.dot_general(
            w65,
            emb_ref[...],
            (((0,), (1,)), ((), ())),
            preferred_element_type=jnp.float32,
        )

    return pl.pallas_call(
        mm,
        grid=(grid,),
        in_specs=[
            pl.BlockSpec((K, TV), lambda i: (0, i)),
            pl.BlockSpec((1, TV), lambda i: (0, i)),
            pl.BlockSpec((B, K1), lambda i: (0, 0)),
        ],
        out_specs=pl.BlockSpec((TV, B), lambda i: (i, 0)),
        out_shape=jax.ShapeDtypeStruct((V, B), jnp.float32),
    )(wt, b2, emb_aug)


def kernel(inputs, emb_table, W, b):
    D = emb_table.shape[1]
    table128 = _tc_transpose_pad(emb_table.T, 128)
    embeds = _sc_gather_sum(inputs.astype(jnp.int32), table128, D)
    emb_aug = jnp.concatenate(
        [embeds, jnp.ones((embeds.shape[0], 1), jnp.float32)], axis=1)
    out_t = _tc_matmul_t(emb_aug, W.T, b.reshape(1, -1))
    return out_t.T
